# baseline (speedup 1.0000x reference)
import jax
import jax.numpy as jnp
from jax import lax
from jax.experimental import pallas as pl
from jax.experimental.pallas import tpu as pltpu

_B = 1024
_D = 64
_V = 100000
_NV = 4096
_G = 8


def _gather_body(x_sref, *refs):
    in_refs = refs[:_G]
    out_ref = refs[_G]
    for g in range(_G):
        out_ref[0, g, :] = in_refs[g][0, 0, :]


def _gather_e(x, w_embed):
    w3 = w_embed.reshape(_V, 1, _D)
    in_specs = [
        pl.BlockSpec(
            (1, 1, _D),
            (lambda g: (lambda i, xs: (xs[i * _G + g], 0, 0)))(g),
        )
        for g in range(_G)
    ]
    grid_spec = pltpu.PrefetchScalarGridSpec(
        num_scalar_prefetch=1,
        grid=(_B // _G,),
        in_specs=in_specs,
        out_specs=pl.BlockSpec((1, _G, _D), lambda i, xs: (i, 0, 0)),
    )
    e3 = pl.pallas_call(
        _gather_body,
        grid_spec=grid_spec,
        out_shape=jax.ShapeDtypeStruct((_B // _G, _G, _D), jnp.float32),
    )(x, *([w3] * _G))
    return e3.reshape(_B, _D)


def _deembed_body(wgt_ref, e_ref, out_ref):
    out_ref[...] = lax.dot_general(
        wgt_ref[...],
        e_ref[...],
        dimension_numbers=(((0,), (1,)), ((), ())),
        preferred_element_type=jnp.float32,
    )


@jax.jit
def kernel(x, w_embed, w_global):
    e = _gather_e(x, w_embed)
    d_t = pl.pallas_call(
        _deembed_body,
        grid=(pl.cdiv(_V, _NV),),
        in_specs=[
            pl.BlockSpec((_D, _NV), lambda i: (0, i)),
            pl.BlockSpec((_B, _D), lambda i: (0, 0)),
        ],
        out_specs=pl.BlockSpec((_NV, _B), lambda i: (i, 0)),
        out_shape=jax.ShapeDtypeStruct((_V, _B), jnp.float32),
    )(w_global.T, e)
    return d_t.T


# speedup vs baseline: 1.1839x; 1.1839x over previous
import jax
import jax.numpy as jnp
from jax import lax
from jax.experimental import pallas as pl
from jax.experimental.pallas import tpu as pltpu

_B = 1024
_D = 64
_V = 100000
_NV = 4096
_G = 32


def _gather_body(x_sref, *refs):
    in_refs = refs[:_G]
    out_ref = refs[_G]
    for g in range(_G):
        out_ref[0, g, :] = in_refs[g][0, 0, :]


def _gather_e(x, w_embed):
    w3 = w_embed.reshape(_V, 1, _D)
    in_specs = [
        pl.BlockSpec(
            (1, 1, _D),
            (lambda g: (lambda i, xs: (xs[i * _G + g], 0, 0)))(g),
        )
        for g in range(_G)
    ]
    grid_spec = pltpu.PrefetchScalarGridSpec(
        num_scalar_prefetch=1,
        grid=(_B // _G,),
        in_specs=in_specs,
        out_specs=pl.BlockSpec((1, _G, _D), lambda i, xs: (i, 0, 0)),
    )
    e3 = pl.pallas_call(
        _gather_body,
        grid_spec=grid_spec,
        out_shape=jax.ShapeDtypeStruct((_B // _G, _G, _D), jnp.float32),
    )(x, *([w3] * _G))
    return e3.reshape(_B, _D)


def _deembed_body(wgt_ref, e_ref, out_ref):
    out_ref[...] = lax.dot_general(
        wgt_ref[...],
        e_ref[...],
        dimension_numbers=(((0,), (1,)), ((), ())),
        preferred_element_type=jnp.float32,
    )


@jax.jit
def kernel(x, w_embed, w_global):
    e = _gather_e(x, w_embed)
    d_t = pl.pallas_call(
        _deembed_body,
        grid=(pl.cdiv(_V, _NV),),
        in_specs=[
            pl.BlockSpec((_D, _NV), lambda i: (0, i)),
            pl.BlockSpec((_B, _D), lambda i: (0, 0)),
        ],
        out_specs=pl.BlockSpec((_NV, _B), lambda i: (i, 0)),
        out_shape=jax.ShapeDtypeStruct((_V, _B), jnp.float32),
    )(w_global.T, e)
    return d_t.T


# speedup vs baseline: 1.1855x; 1.0014x over previous
import jax
import jax.numpy as jnp
from jax import lax
from jax.experimental import pallas as pl
from jax.experimental.pallas import tpu as pltpu

_B = 1024
_D = 64
_V = 100000
_NV = 4096
_G = 64


def _gather_body(x_sref, *refs):
    in_refs = refs[:_G]
    out_ref = refs[_G]
    for g in range(_G):
        out_ref[0, g, :] = in_refs[g][0, 0, :]


def _gather_e(x, w_embed):
    w3 = w_embed.reshape(_V, 1, _D)
    in_specs = [
        pl.BlockSpec(
            (1, 1, _D),
            (lambda g: (lambda i, xs: (xs[i * _G + g], 0, 0)))(g),
        )
        for g in range(_G)
    ]
    grid_spec = pltpu.PrefetchScalarGridSpec(
        num_scalar_prefetch=1,
        grid=(_B // _G,),
        in_specs=in_specs,
        out_specs=pl.BlockSpec((1, _G, _D), lambda i, xs: (i, 0, 0)),
    )
    e3 = pl.pallas_call(
        _gather_body,
        grid_spec=grid_spec,
        out_shape=jax.ShapeDtypeStruct((_B // _G, _G, _D), jnp.float32),
    )(x, *([w3] * _G))
    return e3.reshape(_B, _D)


def _deembed_body(wgt_ref, e_ref, out_ref):
    out_ref[...] = lax.dot_general(
        wgt_ref[...],
        e_ref[...],
        dimension_numbers=(((0,), (1,)), ((), ())),
        preferred_element_type=jnp.float32,
    )


@jax.jit
def kernel(x, w_embed, w_global):
    e = _gather_e(x, w_embed)
    d_t = pl.pallas_call(
        _deembed_body,
        grid=(pl.cdiv(_V, _NV),),
        in_specs=[
            pl.BlockSpec((_D, _NV), lambda i: (0, i)),
            pl.BlockSpec((_B, _D), lambda i: (0, 0)),
        ],
        out_specs=pl.BlockSpec((_NV, _B), lambda i: (i, 0)),
        out_shape=jax.ShapeDtypeStruct((_V, _B), jnp.float32),
    )(w_global.T, e)
    return d_t.T
